# baseline (speedup 1.0000x reference)
import jax
import jax.numpy as jnp
from jax import lax
from jax.experimental import pallas as pl
from jax.experimental.pallas import tpu as pltpu
from jax.experimental.pallas import tpu_sc as plsc

_B = 16384
_C = 5
_D = 64
_NC = 2
_NS = 16
_NW = _NC * _NS
_BPW = _B // _NW
_CB = 128
_NCHUNK = _BPW // _CB
_PAIRS = _CB * _C
_W = 2 * _D
_V = 1000000
_NT = _V // _W
_TPW = _NT // _NW
_NTAIL = _V - _NT * _W


def _transpose_tile(src_view, dst128, inblk, outblk, t):
    iota = lax.iota(jnp.int32, 16)
    pltpu.sync_copy(
        src_view.at[:, pl.ds(pl.multiple_of(t * _W, _W), _W)], inblk)

    def col_pair(ql, carry):
        for m in range(8):
            rows = 16 * (m % 4) + iota
            col = jnp.full((16,), 2 * ql + m // 4, jnp.int32)
            outblk[ql, pl.ds(16 * m, 16)] = plsc.load_gather(
                inblk, [rows, col])
        return carry

    lax.fori_loop(0, _W // 2, col_pair, 0)
    pltpu.sync_copy(
        outblk,
        dst128.at[pl.ds(pl.multiple_of(t * (_W // 2), 8), _W // 2)])


def _k1_body(ctabT_hbm, c128_hbm, inblk, outblk):
    wid = lax.axis_index("s") * _NC + lax.axis_index("c")

    def tile(k, carry):
        _transpose_tile(ctabT_hbm, c128_hbm, inblk, outblk, wid + _NW * k)
        return carry

    lax.fori_loop(0, _TPW, tile, 0)

    @pl.when(wid < _NT - _TPW * _NW)
    def _():
        _transpose_tile(ctabT_hbm, c128_hbm, inblk, outblk,
                        _TPW * _NW + wid)

    @pl.when(wid == 4)
    def _():
        _transpose_tile(ctabT_hbm, c128_hbm, inblk, outblk,
                        lax.min(wid, 0) + _NT)


def _k2_body(tgt_hbm, ctx_hbm, bmap_hbm, ttab_hbm, c128_hbm, out_hbm,
             tidx, cidx, cidx_hi, bmapv, trows, crows, outbuf, sem, gsem):
    wid = lax.axis_index("s") * _NC + lax.axis_index("c")
    base = wid * _BPW
    pltpu.sync_copy(bmap_hbm, bmapv)
    iota = lax.iota(jnp.int32, 16)
    for chunk in range(_NCHUNK):
        b0 = base + chunk * _CB
        pltpu.sync_copy(tgt_hbm.at[pl.ds(b0, _CB)], tidx)
        pltpu.sync_copy(ctx_hbm.at[pl.ds(b0 * _C, _PAIRS)], cidx)

        def prep_c(m, carry):
            v = lax.shift_right_logical(cidx[pl.ds(m * 16, 16)], 1)
            cidx_hi[lax.shift_right_logical(m, 3),
                    pl.ds((m % 8) * 16, 16)] = v
            return carry

        lax.fori_loop(0, _PAIRS // 16, prep_c, 0)

        cps = []
        for j in range(_C):
            cp = pltpu.make_async_copy(c128_hbm.at[cidx_hi.at[j]],
                                       crows.at[pl.ds(j * _CB, _CB)], gsem)
            cp.start()
            cps.append(cp)

        def fire_t(g, carry):
            v = tidx[pl.ds(g * 16, 16)]
            for i in range(16):
                pltpu.make_async_copy(ttab_hbm.at[pl.ds(v[i], 1)],
                                      trows.at[pl.ds(g * 16 + i, 1)],
                                      sem).start()
            return carry

        lax.fori_loop(0, _CB // 16, fire_t, 0)

        def drain_t(k, carry):
            pltpu.make_async_copy(ttab_hbm.at[pl.ds(0, 1)],
                                  trows.at[pl.ds(k, 1)], sem).wait()
            return carry

        lax.fori_loop(0, _CB, drain_t, 0)
        for cp in cps:
            cp.wait()

        def body(g, carry):
            p0 = g * 16
            b_l = bmapv[pl.ds(p0, 16)]
            craw = cidx[pl.ds(p0, 16)]
            cbase = (p0 + iota) * _W + (craw & 1) * _D
            acc = jnp.zeros((16,), jnp.float32)
            for e in range(_D):
                ce = cbase + e
                wv = plsc.load_gather(trows, [b_l, jnp.full((16,), e,
                                                            jnp.int32)])
                cv = plsc.load_gather(crows, [lax.shift_right_logical(ce, 7),
                                              ce & 127])
                acc = acc + wv * cv
            outbuf[pl.ds(p0, 16)] = acc
            return carry

        lax.fori_loop(0, _PAIRS // 16, body, 0)
        pltpu.sync_copy(outbuf, out_hbm.at[pl.ds(b0 * _C, _PAIRS)])


@jax.jit
def kernel(target, context, target_table, context_table):
    tgt = target.astype(jnp.int32)
    ctx = context.reshape(-1).astype(jnp.int32)
    bmap = (jnp.arange(_PAIRS, dtype=jnp.int32) // _C)
    mesh = plsc.VectorSubcoreMesh(core_axis_name="c", subcore_axis_name="s",
                                  num_cores=_NC, num_subcores=_NS)
    params = pltpu.CompilerParams(needs_layout_passes=False,
                                  use_tc_tiling_on_sc=True)
    c128 = pl.kernel(
        _k1_body,
        out_type=jax.ShapeDtypeStruct((_V // 2 + 32, _W), jnp.float32),
        mesh=mesh,
        compiler_params=params,
        scratch_types=[
            pltpu.VMEM((_D, _W), jnp.float32),
            pltpu.VMEM((_D, _W), jnp.float32),
        ],
    )(context_table.T)
    out_flat = pl.kernel(
        _k2_body,
        out_type=jax.ShapeDtypeStruct((_B * _C,), jnp.float32),
        mesh=mesh,
        compiler_params=params,
        scratch_types=[
            pltpu.VMEM((_CB,), jnp.int32),
            pltpu.VMEM((_PAIRS,), jnp.int32),
            pltpu.VMEM((_C, _CB), jnp.int32),
            pltpu.VMEM((_PAIRS,), jnp.int32),
            pltpu.VMEM((_CB, _D), jnp.float32),
            pltpu.VMEM((_PAIRS, _W), jnp.float32),
            pltpu.VMEM((_PAIRS,), jnp.float32),
            pltpu.SemaphoreType.DMA,
            pltpu.SemaphoreType.DMA,
        ],
    )(tgt, ctx, bmap, target_table, c128)
    return out_flat.reshape(_B, _C)


# speedup vs baseline: 2.0433x; 2.0433x over previous
import jax
import jax.numpy as jnp
from jax import lax
from jax.experimental import pallas as pl
from jax.experimental.pallas import tpu as pltpu
from jax.experimental.pallas import tpu_sc as plsc

_B = 16384
_C = 5
_D = 64
_NC = 2
_NS = 16
_NW = _NC * _NS
_BPW = _B // _NW
_CB = 128
_NCHUNK = _BPW // _CB
_PAIRS = _CB * _C
_W = 2 * _D
_V = 1000000
_NT = _V // _W
_TPW = _NT // _NW
_NTAIL = _V - _NT * _W


def _transpose_blk(inblk, outblk):
    iota = lax.iota(jnp.int32, 16)

    @plsc.parallel_loop(0, _W // 2, unroll=8)
    def col_pair(ql):
        for m in range(8):
            rows = 16 * (m % 4) + iota
            col = jnp.full((16,), 2 * ql + m // 4, jnp.int32)
            outblk[ql, pl.ds(16 * m, 16)] = plsc.load_gather(
                inblk, [rows, col])


def _in_cp(src_view, t, inblk, sem):
    return pltpu.make_async_copy(
        src_view.at[:, pl.ds(pl.multiple_of(t * _W, _W), _W)], inblk, sem)


def _out_cp(dst128, t, outblk, sem):
    return pltpu.make_async_copy(
        outblk, dst128.at[pl.ds(pl.multiple_of(t * (_W // 2), 8), _W // 2)],
        sem)


def _transpose_tile(src_view, dst128, inblk, outblk, t, sem):
    _in_cp(src_view, t, inblk, sem).start()
    _in_cp(src_view, t, inblk, sem).wait()
    _transpose_blk(inblk, outblk)
    _out_cp(dst128, t, outblk, sem).start()
    _out_cp(dst128, t, outblk, sem).wait()


def _k1_body(ctabT_hbm, c128_hbm, in0, in1, out0, out1,
             sin0, sin1, sout0, sout1, sem):
    wid = lax.axis_index("s") * _NC + lax.axis_index("c")
    ins = (in0, in1)
    outs = (out0, out1)
    sis = (sin0, sin1)
    sos = (sout0, sout1)
    _in_cp(ctabT_hbm, wid, in0, sin0).start()

    def pair(j, carry):
        t0 = wid + _NW * 2 * j
        t1 = t0 + _NW
        tp = lax.min(t1 + _NW, _NT)
        _in_cp(ctabT_hbm, t1, in1, sin1).start()
        _in_cp(ctabT_hbm, t0, in0, sin0).wait()

        @pl.when(j > 0)
        def _():
            _out_cp(c128_hbm, t0, out0, sout0).wait()

        _transpose_blk(in0, out0)
        _out_cp(c128_hbm, t0, out0, sout0).start()

        _in_cp(ctabT_hbm, tp, in0, sin0).start()
        _in_cp(ctabT_hbm, t1, in1, sin1).wait()

        @pl.when(j > 0)
        def _():
            _out_cp(c128_hbm, t1, out1, sout1).wait()

        _transpose_blk(in1, out1)
        _out_cp(c128_hbm, t1, out1, sout1).start()
        return carry

    lax.fori_loop(0, _TPW // 2, pair, 0)
    tlast = wid + _NW * (_TPW - 1)
    _out_cp(c128_hbm, tlast, out0, sout0).wait()
    _out_cp(c128_hbm, tlast, out1, sout1).wait()
    _in_cp(ctabT_hbm, lax.min(wid + _NW * _TPW, _NT), in0, sin0).wait()

    @pl.when(wid < _NT - _TPW * _NW)
    def _():
        _transpose_tile(ctabT_hbm, c128_hbm, in0, out0,
                        _TPW * _NW + wid, sem)

    @pl.when(wid == 4)
    def _():
        _transpose_tile(ctabT_hbm, c128_hbm, in0, out0,
                        lax.min(wid, 0) + _NT, sem)


def _k2_body(tgt_hbm, ctx_hbm, bmap_hbm, ttab_hbm, c128_hbm, out_hbm,
             tidx, cidx, cidx_hi, bmapv, trows, crows, outbuf, sem, gsem):
    wid = lax.axis_index("s") * _NC + lax.axis_index("c")
    base = wid * _BPW
    pltpu.sync_copy(bmap_hbm, bmapv)
    iota = lax.iota(jnp.int32, 16)
    for chunk in range(_NCHUNK):
        b0 = base + chunk * _CB
        pltpu.sync_copy(tgt_hbm.at[pl.ds(b0, _CB)], tidx)
        pltpu.sync_copy(ctx_hbm.at[pl.ds(b0 * _C, _PAIRS)], cidx)

        def prep_c(m, carry):
            v = lax.shift_right_logical(cidx[pl.ds(m * 16, 16)], 1)
            cidx_hi[lax.shift_right_logical(m, 3),
                    pl.ds((m % 8) * 16, 16)] = v
            return carry

        lax.fori_loop(0, _PAIRS // 16, prep_c, 0)

        cps = []
        for j in range(_C):
            cp = pltpu.make_async_copy(c128_hbm.at[cidx_hi.at[j]],
                                       crows.at[pl.ds(j * _CB, _CB)], gsem)
            cp.start()
            cps.append(cp)

        def fire_t(g, carry):
            v = tidx[pl.ds(g * 16, 16)]
            for i in range(16):
                pltpu.make_async_copy(ttab_hbm.at[pl.ds(v[i], 1)],
                                      trows.at[pl.ds(g * 16 + i, 1)],
                                      sem).start()
            return carry

        lax.fori_loop(0, _CB // 16, fire_t, 0)

        def drain_t(k, carry):
            pltpu.make_async_copy(ttab_hbm.at[pl.ds(0, 1)],
                                  trows.at[pl.ds(k, 1)], sem).wait()
            return carry

        lax.fori_loop(0, _CB, drain_t, 0)
        for cp in cps:
            cp.wait()

        def body(g, carry):
            p0 = g * 16
            b_l = bmapv[pl.ds(p0, 16)]
            craw = cidx[pl.ds(p0, 16)]
            cbase = (p0 + iota) * _W + (craw & 1) * _D
            acc = jnp.zeros((16,), jnp.float32)
            for e in range(_D):
                ce = cbase + e
                wv = plsc.load_gather(trows, [b_l, jnp.full((16,), e,
                                                            jnp.int32)])
                cv = plsc.load_gather(crows, [lax.shift_right_logical(ce, 7),
                                              ce & 127])
                acc = acc + wv * cv
            outbuf[pl.ds(p0, 16)] = acc
            return carry

        lax.fori_loop(0, _PAIRS // 16, body, 0)
        pltpu.sync_copy(outbuf, out_hbm.at[pl.ds(b0 * _C, _PAIRS)])


@jax.jit
def kernel(target, context, target_table, context_table):
    tgt = target.astype(jnp.int32)
    ctx = context.reshape(-1).astype(jnp.int32)
    bmap = (jnp.arange(_PAIRS, dtype=jnp.int32) // _C)
    mesh = plsc.VectorSubcoreMesh(core_axis_name="c", subcore_axis_name="s",
                                  num_cores=_NC, num_subcores=_NS)
    params = pltpu.CompilerParams(needs_layout_passes=False,
                                  use_tc_tiling_on_sc=True)
    c128 = pl.kernel(
        _k1_body,
        out_type=jax.ShapeDtypeStruct((_V // 2 + 32, _W), jnp.float32),
        mesh=mesh,
        compiler_params=params,
        scratch_types=[
            pltpu.VMEM((_D, _W), jnp.float32),
            pltpu.VMEM((_D, _W), jnp.float32),
            pltpu.VMEM((_D, _W), jnp.float32),
            pltpu.VMEM((_D, _W), jnp.float32),
            pltpu.SemaphoreType.DMA,
            pltpu.SemaphoreType.DMA,
            pltpu.SemaphoreType.DMA,
            pltpu.SemaphoreType.DMA,
            pltpu.SemaphoreType.DMA,
        ],
    )(context_table.T)
    out_flat = pl.kernel(
        _k2_body,
        out_type=jax.ShapeDtypeStruct((_B * _C,), jnp.float32),
        mesh=mesh,
        compiler_params=params,
        scratch_types=[
            pltpu.VMEM((_CB,), jnp.int32),
            pltpu.VMEM((_PAIRS,), jnp.int32),
            pltpu.VMEM((_C, _CB), jnp.int32),
            pltpu.VMEM((_PAIRS,), jnp.int32),
            pltpu.VMEM((_CB, _D), jnp.float32),
            pltpu.VMEM((_PAIRS, _W), jnp.float32),
            pltpu.VMEM((_PAIRS,), jnp.float32),
            pltpu.SemaphoreType.DMA,
            pltpu.SemaphoreType.DMA,
        ],
    )(tgt, ctx, bmap, target_table, c128)
    return out_flat.reshape(_B, _C)
